# Initial kernel scaffold; baseline (speedup 1.0000x reference)
import functools

import jax
import jax.numpy as jnp
from jax import lax
from jax.experimental import pallas as pl
from jax.experimental.pallas import tpu as pltpu
from jax.experimental.pallas import tpu_sc as plsc

B, L, C = 16, 4096, 256
SIZE2 = 2
LOUT = L * SIZE2
NW = 32
CPT = C // NW
CHUNK = 1024
NCHUNK = L // CHUNK
LANES = 16


def _build_kernel():
  mesh = plsc.VectorSubcoreMesh(core_axis_name="c", subcore_axis_name="s")

  @functools.partial(
      pl.kernel,
      out_type=jax.ShapeDtypeStruct((B, LOUT, C), jnp.float32),
      mesh=mesh,
      scratch_types=[
          pltpu.VMEM((CHUNK, CPT), jnp.int32),
          pltpu.VMEM((CHUNK, CPT), jnp.float32),
          pltpu.VMEM((LOUT, CPT), jnp.float32),
      ],
  )
  def unpool(upd_hbm, mask_hbm, out_hbm, m_v, u_v, acc_v):
    wid = lax.axis_index("s") * 2 + lax.axis_index("c")
    cc0 = wid * CPT
    lane = lax.iota(jnp.int32, LANES)
    cc_pat = lane % CPT
    zero = jnp.zeros((LANES,), jnp.float32)
    accf = acc_v.reshape(LOUT * CPT)
    mf = m_v.reshape(CHUNK * CPT)
    uf = u_v.reshape(CHUNK * CPT)

    def batch_body(b, _):
      def zero_body(i, _):
        accf[pl.ds(i * LANES, LANES)] = zero
        return ()
      lax.fori_loop(0, LOUT * CPT // LANES, zero_body, ())

      for k in range(NCHUNK):
        pltpu.sync_copy(
            mask_hbm.at[b, pl.ds(k * CHUNK, CHUNK), pl.ds(cc0, CPT)], m_v)
        pltpu.sync_copy(
            upd_hbm.at[b, pl.ds(k * CHUNK, CHUNK), pl.ds(cc0, CPT)], u_v)

        def chunk_body(i, _):
          m = mf[pl.ds(i * LANES, LANES)]
          u = uf[pl.ds(i * LANES, LANES)]
          x = lax.shift_right_logical(m, 8)
          plsc.addupdate_scatter(acc_v, [x, cc_pat], u)
          return ()
        lax.fori_loop(0, CHUNK * CPT // LANES, chunk_body, ())

      pltpu.sync_copy(acc_v, out_hbm.at[b, :, pl.ds(cc0, CPT)])
      return ()

    lax.fori_loop(0, B, batch_body, ())

  return unpool


_unpool = _build_kernel()


def kernel(updates, mask):
  return _unpool(updates, mask.astype(jnp.int32))


# speedup vs baseline: 39.6726x; 39.6726x over previous
import functools

import jax
import jax.numpy as jnp
from jax import lax
from jax.experimental import pallas as pl
from jax.experimental.pallas import tpu as pltpu
from jax.experimental.pallas import tpu_sc as plsc

B, L, C = 16, 4096, 256
SIZE2 = 2
LOUT = L * SIZE2
NW = 32
CPT = C // NW
CHUNK = 1024
NCHUNK = L // CHUNK
LANES = 16


def _build_kernel():
  mesh = plsc.VectorSubcoreMesh(core_axis_name="c", subcore_axis_name="s")

  @functools.partial(
      pl.kernel,
      out_type=jax.ShapeDtypeStruct((B, LOUT, C), jnp.float32),
      mesh=mesh,
      scratch_types=[
          pltpu.VMEM((CHUNK, CPT), jnp.int32),
          pltpu.VMEM((CHUNK, CPT), jnp.float32),
          pltpu.VMEM((LOUT, CPT), jnp.float32),
      ],
      compiler_params=pltpu.CompilerParams(
          use_tc_tiling_on_sc=False, needs_layout_passes=False),
  )
  def unpool(upd_hbm, mask_hbm, out_hbm, m_v, u_v, acc_v):
    wid = lax.axis_index("s") * 2 + lax.axis_index("c")
    cc0 = wid * CPT
    lane = lax.iota(jnp.int32, LANES)
    cc_pat = lane % CPT
    row_off = lane // CPT
    zero = jnp.zeros((LANES,), jnp.float32)

    def batch_body(b, _):
      def zero_body(i, _):
        plsc.store_scatter(acc_v, [2 * i + row_off, cc_pat], zero)
        return ()
      lax.fori_loop(0, LOUT * CPT // LANES, zero_body, ())

      for k in range(NCHUNK):
        pltpu.sync_copy(
            mask_hbm.at[b, pl.ds(k * CHUNK, CHUNK), pl.ds(cc0, CPT)], m_v)
        pltpu.sync_copy(
            upd_hbm.at[b, pl.ds(k * CHUNK, CHUNK), pl.ds(cc0, CPT)], u_v)

        def chunk_body(i, _):
          rows = 2 * i + row_off
          m = plsc.load_gather(m_v, [rows, cc_pat])
          u = plsc.load_gather(u_v, [rows, cc_pat])
          x = lax.shift_right_logical(m, 8)
          plsc.addupdate_scatter(acc_v, [x, cc_pat], u)
          return ()
        lax.fori_loop(0, CHUNK * CPT // LANES, chunk_body, ())

      pltpu.sync_copy(acc_v, out_hbm.at[b, :, pl.ds(cc0, CPT)])
      return ()

    lax.fori_loop(0, B, batch_body, ())

  return unpool


_unpool = _build_kernel()


def kernel(updates, mask):
  return _unpool(updates, mask.astype(jnp.int32))


# speedup vs baseline: 47.2832x; 1.1918x over previous
import functools

import jax
import jax.numpy as jnp
from jax import lax
from jax.experimental import pallas as pl
from jax.experimental.pallas import tpu as pltpu
from jax.experimental.pallas import tpu_sc as plsc

B, L, C = 16, 4096, 256
SIZE2 = 2
LOUT = L * SIZE2
NW = 32
CPT = C // NW
CHUNK = 1024
NCHUNK = L // CHUNK
LANES = 16


def _build_kernel():
  mesh = plsc.VectorSubcoreMesh(core_axis_name="c", subcore_axis_name="s")

  @functools.partial(
      pl.kernel,
      out_type=jax.ShapeDtypeStruct((B, LOUT, C), jnp.float32),
      mesh=mesh,
      scratch_types=[
          pltpu.VMEM((CHUNK, CPT), jnp.int32),
          pltpu.VMEM((CHUNK, CPT), jnp.float32),
          pltpu.VMEM((LOUT, CPT), jnp.float32),
      ],
      compiler_params=pltpu.CompilerParams(
          use_tc_tiling_on_sc=False, needs_layout_passes=False),
  )
  def unpool(upd_hbm, mask_hbm, out_hbm, m_v, u_v, acc_v):
    wid = lax.axis_index("s") * 2 + lax.axis_index("c")
    cc0 = wid * CPT
    lane = lax.iota(jnp.int32, LANES)
    cc_pat = lane % CPT
    row_off = lane // CPT
    zero = jnp.zeros((LANES,), jnp.float32)

    def batch_body(b, _):
      def zero_body(i, _):
        plsc.store_scatter(acc_v, [2 * i + row_off, cc_pat], zero)
        return ()
      lax.fori_loop(0, LOUT * CPT // LANES, zero_body, (), unroll=8)

      for k in range(NCHUNK):
        pltpu.sync_copy(
            mask_hbm.at[b, pl.ds(k * CHUNK, CHUNK), pl.ds(cc0, CPT)], m_v)
        pltpu.sync_copy(
            upd_hbm.at[b, pl.ds(k * CHUNK, CHUNK), pl.ds(cc0, CPT)], u_v)

        def chunk_body(i, _):
          rows = 2 * i + row_off
          m = plsc.load_gather(m_v, [rows, cc_pat])
          u = plsc.load_gather(u_v, [rows, cc_pat])
          x = lax.shift_right_logical(m, 8)
          plsc.addupdate_scatter(acc_v, [x, cc_pat], u)
          return ()
        lax.fori_loop(0, CHUNK * CPT // LANES, chunk_body, (), unroll=8)

      pltpu.sync_copy(acc_v, out_hbm.at[b, :, pl.ds(cc0, CPT)])
      return ()

    lax.fori_loop(0, B, batch_body, ())

  return unpool


_unpool = _build_kernel()


def kernel(updates, mask):
  return _unpool(updates, mask.astype(jnp.int32))


# speedup vs baseline: 59.0491x; 1.2488x over previous
import functools

import jax
import jax.numpy as jnp
from jax import lax
from jax.experimental import pallas as pl
from jax.experimental.pallas import tpu as pltpu
from jax.experimental.pallas import tpu_sc as plsc

B, L, C = 16, 4096, 256
SIZE2 = 2
LOUT = L * SIZE2
NW = 32
CPT = C // NW
CHUNK = 1024
NCHUNK = L // CHUNK
LANES = 16


def _build_kernel():
  mesh = plsc.VectorSubcoreMesh(core_axis_name="c", subcore_axis_name="s")

  @functools.partial(
      pl.kernel,
      out_type=jax.ShapeDtypeStruct((B, LOUT, C), jnp.float32),
      mesh=mesh,
      scratch_types=[
          pltpu.VMEM((CHUNK, CPT), jnp.int32),
          pltpu.VMEM((CHUNK, CPT), jnp.float32),
          pltpu.VMEM((LOUT, CPT), jnp.float32),
      ],
      compiler_params=pltpu.CompilerParams(
          use_tc_tiling_on_sc=False, needs_layout_passes=False),
  )
  def unpool(upd_hbm, mask_hbm, out_hbm, m_v, u_v, acc_v):
    wid = lax.axis_index("s") * 2 + lax.axis_index("c")
    cc0 = wid * CPT
    lane = lax.iota(jnp.int32, LANES)
    cc_pat = lane % CPT
    row_off = lane // CPT
    zero = jnp.zeros((LANES,), jnp.float32)

    def batch_body(b, _):
      @plsc.parallel_loop(0, LOUT * CPT // LANES, unroll=8)
      def _(i):
        plsc.store_scatter(acc_v, [2 * i + row_off, cc_pat], zero)

      for k in range(NCHUNK):
        pltpu.sync_copy(
            mask_hbm.at[b, pl.ds(k * CHUNK, CHUNK), pl.ds(cc0, CPT)], m_v)
        pltpu.sync_copy(
            upd_hbm.at[b, pl.ds(k * CHUNK, CHUNK), pl.ds(cc0, CPT)], u_v)

        @plsc.parallel_loop(0, CHUNK * CPT // LANES, unroll=8)
        def _(i):
          rows = 2 * i + row_off
          m = plsc.load_gather(m_v, [rows, cc_pat])
          u = plsc.load_gather(u_v, [rows, cc_pat])
          x = lax.shift_right_logical(m, 8)
          plsc.addupdate_scatter(acc_v, [x, cc_pat], u)

      pltpu.sync_copy(acc_v, out_hbm.at[b, :, pl.ds(cc0, CPT)])
      return ()

    lax.fori_loop(0, B, batch_body, ())

  return unpool


_unpool = _build_kernel()


def kernel(updates, mask):
  return _unpool(updates, mask.astype(jnp.int32))


# speedup vs baseline: 77.3642x; 1.3102x over previous
import functools

import jax
import jax.numpy as jnp
from jax import lax
from jax.experimental import pallas as pl
from jax.experimental.pallas import tpu as pltpu
from jax.experimental.pallas import tpu_sc as plsc

B, L, C = 16, 4096, 256
SIZE2 = 2
LOUT = L * SIZE2
NW = 32
CPT = C // NW
CHUNK = 1024
NCHUNK = L // CHUNK
LANES = 16


def _build_kernel():
  mesh = plsc.VectorSubcoreMesh(core_axis_name="c", subcore_axis_name="s")

  @functools.partial(
      pl.kernel,
      out_type=jax.ShapeDtypeStruct((B, LOUT, C), jnp.float32),
      mesh=mesh,
      scratch_types=[
          pltpu.VMEM((2, CHUNK, CPT), jnp.int32),
          pltpu.VMEM((2, CHUNK, CPT), jnp.float32),
          pltpu.VMEM((LOUT, CPT), jnp.float32),
          pltpu.SemaphoreType.DMA,
          pltpu.SemaphoreType.DMA,
      ],
      compiler_params=pltpu.CompilerParams(
          use_tc_tiling_on_sc=False, needs_layout_passes=False),
  )
  def unpool(upd_hbm, mask_hbm, out_hbm, m_v, u_v, acc_v, in_sem, out_sem):
    wid = lax.axis_index("s") * 2 + lax.axis_index("c")
    cc0 = wid * CPT
    lane = lax.iota(jnp.int32, LANES)
    cc_pat = lane % CPT
    row_off = lane // CPT
    zero = jnp.zeros((LANES,), jnp.float32)

    def start_in(bb, kk, buf):
      r0 = kk * CHUNK
      pltpu.make_async_copy(
          mask_hbm.at[bb, pl.ds(r0, CHUNK), pl.ds(cc0, CPT)],
          m_v.at[buf], in_sem).start()
      pltpu.make_async_copy(
          upd_hbm.at[bb, pl.ds(r0, CHUNK), pl.ds(cc0, CPT)],
          u_v.at[buf], in_sem).start()

    def wait_in(bb, kk, buf):
      pltpu.make_async_copy(
          mask_hbm.at[bb, pl.ds(kk * CHUNK, CHUNK), pl.ds(cc0, CPT)],
          m_v.at[buf], in_sem).wait()
      pltpu.make_async_copy(
          upd_hbm.at[bb, pl.ds(kk * CHUNK, CHUNK), pl.ds(cc0, CPT)],
          u_v.at[buf], in_sem).wait()

    def out_desc(bb):
      return pltpu.make_async_copy(
          acc_v, out_hbm.at[bb, :, pl.ds(cc0, CPT)], out_sem)

    start_in(0, 0, 0)
    start_in(0, 1, 1)

    def batch_body(b, _):
      @pl.when(b > 0)
      def _():
        out_desc(jnp.maximum(b - 1, 0)).wait()

      @plsc.parallel_loop(0, LOUT * CPT // LANES, unroll=8)
      def _(i):
        plsc.store_scatter(acc_v, [2 * i + row_off, cc_pat], zero)

      for k in range(NCHUNK):
        buf = k % 2
        wait_in(b, k, buf)

        @plsc.parallel_loop(0, CHUNK * CPT // LANES, unroll=8)
        def _(i):
          rows = 2 * i + row_off
          m = plsc.load_gather(m_v.at[buf], [rows, cc_pat])
          u = plsc.load_gather(u_v.at[buf], [rows, cc_pat])
          x = lax.shift_right_logical(m, 8)
          plsc.addupdate_scatter(acc_v, [x, cc_pat], u)

        nk = k + 2
        bb = jnp.minimum(b + nk // NCHUNK, B - 1)
        start_in(bb, nk % NCHUNK, nk % 2)

      out_desc(b).start()
      return ()

    lax.fori_loop(0, B, batch_body, ())

    out_desc(B - 1).wait()
    wait_in(B - 1, 0, 0)
    wait_in(B - 1, 1, 1)

  return unpool


_unpool = _build_kernel()


def kernel(updates, mask):
  return _unpool(updates, mask.astype(jnp.int32))


# speedup vs baseline: 112.4549x; 1.4536x over previous
import functools

import jax
import jax.numpy as jnp
from jax import lax
from jax.experimental import pallas as pl
from jax.experimental.pallas import tpu as pltpu
from jax.experimental.pallas import tpu_sc as plsc

B, L, C = 16, 4096, 256
SIZE2 = 2
LOUT = L * SIZE2
CPT = 8
CHUNK = 1024
NCHUNK = L // CHUNK
CB = CHUNK // 8
LANES = 16
NBLK_IN = L // 8
NBLK_OUT = LOUT // 8


def _build_kernel():
  mesh = plsc.VectorSubcoreMesh(core_axis_name="c", subcore_axis_name="s")

  @functools.partial(
      pl.kernel,
      out_type=jax.ShapeDtypeStruct((B * NBLK_OUT, 2, 8, 128), jnp.float32),
      mesh=mesh,
      scratch_types=[
          pltpu.VMEM((2, CB, 8, CPT), jnp.int32),
          pltpu.VMEM((2, CB, 8, CPT), jnp.float32),
          pltpu.VMEM((NBLK_OUT, 8, CPT), jnp.float32),
          pltpu.SemaphoreType.DMA,
          pltpu.SemaphoreType.DMA,
      ],
      compiler_params=pltpu.CompilerParams(
          use_tc_tiling_on_sc=False, needs_layout_passes=False),
  )
  def unpool(upd_hbm, mask_hbm, out_hbm, m_v, u_v, acc_v, in_sem, out_sem):
    cb = lax.axis_index("c")
    sid = lax.axis_index("s")
    cc0 = sid * CPT
    lane = lax.iota(jnp.int32, LANES)
    cc_pat = lane % CPT
    row_off = lane // CPT
    zero = jnp.zeros((LANES,), jnp.float32)

    def in_desc(bb, kk, buf):
      bl0 = bb * NBLK_IN + kk * CB
      return (
          pltpu.make_async_copy(
              mask_hbm.at[pl.ds(bl0, CB), cb, :, pl.ds(cc0, CPT)],
              m_v.at[buf], in_sem),
          pltpu.make_async_copy(
              upd_hbm.at[pl.ds(bl0, CB), cb, :, pl.ds(cc0, CPT)],
              u_v.at[buf], in_sem),
      )

    def start_in(bb, kk, buf):
      for d in in_desc(bb, kk, buf):
        d.start()

    def wait_in(bb, kk, buf):
      for d in in_desc(bb, kk, buf):
        d.wait()

    def out_desc(bb):
      return pltpu.make_async_copy(
          acc_v,
          out_hbm.at[pl.ds(bb * NBLK_OUT, NBLK_OUT), cb, :, pl.ds(cc0, CPT)],
          out_sem)

    start_in(0, 0, 0)
    start_in(0, 1, 1)

    def batch_body(b, _):
      @pl.when(b > 0)
      def _():
        out_desc(jnp.maximum(b - 1, 0)).wait()

      @plsc.parallel_loop(0, NBLK_OUT * 8 * CPT // LANES, unroll=8)
      def _(i):
        q = jnp.broadcast_to(lax.shift_right_logical(i, 2), (LANES,))
        r8 = ((i & 3) << 1) + row_off
        plsc.store_scatter(acc_v, [q, r8, cc_pat], zero)

      for k in range(NCHUNK):
        buf = k % 2
        wait_in(b, k, buf)

        @plsc.parallel_loop(0, CB * 8 * CPT // LANES, unroll=8)
        def _(i):
          q = jnp.broadcast_to(lax.shift_right_logical(i, 2), (LANES,))
          r8 = ((i & 3) << 1) + row_off
          m = plsc.load_gather(m_v.at[buf], [q, r8, cc_pat])
          u = plsc.load_gather(u_v.at[buf], [q, r8, cc_pat])
          x = lax.shift_right_logical(m, 8)
          plsc.addupdate_scatter(
              acc_v, [lax.shift_right_logical(x, 3), x & 7, cc_pat], u)

        nk = k + 2
        bb = jnp.minimum(b + nk // NCHUNK, B - 1)
        start_in(bb, nk % NCHUNK, nk % 2)

      out_desc(b).start()
      return ()

    lax.fori_loop(0, B, batch_body, ())

    out_desc(B - 1).wait()
    wait_in(B - 1, 0, 0)
    wait_in(B - 1, 1, 1)

  return unpool


_unpool = _build_kernel()


def kernel(updates, mask):
  u4 = (updates.reshape(B, NBLK_IN, 8, 2, 128)
        .transpose(0, 1, 3, 2, 4).reshape(B * NBLK_IN, 2, 8, 128))
  m4 = (mask.astype(jnp.int32).reshape(B, NBLK_IN, 8, 2, 128)
        .transpose(0, 1, 3, 2, 4).reshape(B * NBLK_IN, 2, 8, 128))
  o4 = _unpool(u4, m4)
  return (o4.reshape(B, NBLK_OUT, 2, 8, 128)
          .transpose(0, 1, 3, 2, 4).reshape(B, LOUT, C))


# speedup vs baseline: 122.9454x; 1.0933x over previous
import functools

import jax
import jax.numpy as jnp
from jax import lax
from jax.experimental import pallas as pl
from jax.experimental.pallas import tpu as pltpu
from jax.experimental.pallas import tpu_sc as plsc

B, L, C = 16, 4096, 256
SIZE2 = 2
LOUT = L * SIZE2
CPT = 16
CHUNK = 512
NCHUNK = L // CHUNK
CB = CHUNK // 8
LANES = 16
NBLK_IN = L // 8
NBLK_OUT = LOUT // 8
HB = LOUT // 2 // 8


def _build_kernel():
  mesh = plsc.VectorSubcoreMesh(core_axis_name="c", subcore_axis_name="s")

  @functools.partial(
      pl.kernel,
      out_type=jax.ShapeDtypeStruct((B * NBLK_OUT, 2, 8, 128), jnp.float32),
      mesh=mesh,
      scratch_types=[
          pltpu.VMEM((2, CB, 8, CPT), jnp.int32),
          pltpu.VMEM((2, CB, 8, CPT), jnp.float32),
          pltpu.VMEM((HB, 8, CPT), jnp.float32),
          pltpu.SemaphoreType.DMA,
          pltpu.SemaphoreType.DMA,
      ],
      compiler_params=pltpu.CompilerParams(
          use_tc_tiling_on_sc=False, needs_layout_passes=False),
  )
  def unpool(upd_hbm, mask_hbm, out_hbm, m_v, u_v, acc_v, in_sem, out_sem):
    h = lax.axis_index("c")
    sid = lax.axis_index("s")
    cbg = lax.shift_right_logical(sid, 3)
    ch0 = (sid & 7) * CPT
    lane = lax.iota(jnp.int32, LANES)
    zero = jnp.zeros((LANES,), jnp.float32)

    def in_desc(bb, kk, buf):
      bl0 = bb * NBLK_IN + kk * CB
      return (
          pltpu.make_async_copy(
              mask_hbm.at[pl.ds(bl0, CB), cbg, :, pl.ds(ch0, CPT)],
              m_v.at[buf], in_sem),
          pltpu.make_async_copy(
              upd_hbm.at[pl.ds(bl0, CB), cbg, :, pl.ds(ch0, CPT)],
              u_v.at[buf], in_sem),
      )

    def start_in(bb, kk, buf):
      for d in in_desc(bb, kk, buf):
        d.start()

    def wait_in(bb, kk, buf):
      for d in in_desc(bb, kk, buf):
        d.wait()

    def out_desc(bb):
      return pltpu.make_async_copy(
          acc_v,
          out_hbm.at[pl.ds(bb * NBLK_OUT + h * HB, HB), cbg, :,
                     pl.ds(ch0, CPT)],
          out_sem)

    start_in(0, 0, 0)
    start_in(0, 1, 1)

    def batch_body(b, _):
      @pl.when(b > 0)
      def _():
        out_desc(jnp.maximum(b - 1, 0)).wait()

      @plsc.parallel_loop(0, HB * 8, unroll=8)
      def _(i):
        acc_v[lax.shift_right_logical(i, 3), i & 7, :] = zero

      for k in range(NCHUNK):
        buf = k % 2
        wait_in(b, k, buf)

        @plsc.parallel_loop(0, CB * 8, unroll=8)
        def _(i):
          q = lax.shift_right_logical(i, 3)
          r8 = i & 7
          m = m_v[buf, q, r8, :]
          u = u_v[buf, q, r8, :]
          x = lax.shift_right_logical(m, 8)
          keep = lax.shift_right_logical(x, 12) == h
          xl = x & (LOUT // 2 - 1)
          plsc.addupdate_scatter(
              acc_v, [lax.shift_right_logical(xl, 3), xl & 7, lane], u,
              mask=keep)

        nk = k + 2
        bb = jnp.minimum(b + nk // NCHUNK, B - 1)
        start_in(bb, nk % NCHUNK, nk % 2)

      out_desc(b).start()
      return ()

    lax.fori_loop(0, B, batch_body, ())

    out_desc(B - 1).wait()
    wait_in(B - 1, 0, 0)
    wait_in(B - 1, 1, 1)

  return unpool


_unpool = _build_kernel()


def kernel(updates, mask):
  u4 = (updates.reshape(B, NBLK_IN, 8, 2, 128)
        .transpose(0, 1, 3, 2, 4).reshape(B * NBLK_IN, 2, 8, 128))
  m4 = (mask.astype(jnp.int32).reshape(B, NBLK_IN, 8, 2, 128)
        .transpose(0, 1, 3, 2, 4).reshape(B * NBLK_IN, 2, 8, 128))
  o4 = _unpool(u4, m4)
  return (o4.reshape(B, NBLK_OUT, 2, 8, 128)
          .transpose(0, 1, 3, 2, 4).reshape(B, LOUT, C))
